# Initial kernel scaffold; baseline (speedup 1.0000x reference)
import jax
import jax.numpy as jnp
from jax.experimental import pallas as pl


def kernel(x_res, x_AA, edge_index_backbone, edge_index_contact, params):
    raise NotImplementedError("write your pallas kernel here")


# speedup vs baseline: 4.1296x; 4.1296x over previous
import functools

import jax
import jax.numpy as jnp
from jax import lax
from jax.experimental import pallas as pl
from jax.experimental.pallas import tpu as pltpu
from jax.experimental.pallas import tpu_sc as plsc

_N_TILES = 16
_CH = 80
_BLK = 2000
_EPS = 1e-5


def _gelu(x):
    return 0.5 * x * (1.0 + lax.erf(x * (2.0 ** -0.5)))


def _make_sc_agg(NPAD, F, E, with_counts):
    EPT = E // _N_TILES
    NCH = EPT // _CH
    RPT = NPAD // _N_TILES
    mesh = plsc.VectorSubcoreMesh(core_axis_name="c", subcore_axis_name="s")

    out_type = [jax.ShapeDtypeStruct((2, NPAD, F), jnp.float32)]
    scratch = [
        pltpu.VMEM((1, _CH), jnp.int32),
        pltpu.VMEM((1, _CH), jnp.int32),
        pltpu.VMEM((_CH, F), jnp.float32),
        pltpu.VMEM_SHARED((NPAD, F), jnp.float32),
        pltpu.SemaphoreType.DMA,
    ]
    if with_counts:
        out_type.append(jax.ShapeDtypeStruct((2, NPAD, 16), jnp.float32))
        scratch.append(pltpu.VMEM((_CH, 16), jnp.float32))
        scratch.append(pltpu.VMEM_SHARED((NPAD, 16), jnp.float32))

    def body(h_hbm, ei_hbm, z_hbm, *rest):
        if with_counts:
            (out_sums, out_cnts, idx_s, idx_d, rows, acc, sem,
             ones_v, cnt) = rest
        else:
            out_sums, idx_s, idx_d, rows, acc, sem = rest
        c = lax.axis_index("c")
        s = lax.axis_index("s")
        r0 = s * RPT
        pltpu.sync_copy(z_hbm.at[pl.ds(r0, RPT)], acc.at[pl.ds(r0, RPT)])
        if with_counts:
            pltpu.sync_copy(z_hbm.at[pl.ds(r0, RPT), pl.ds(0, 16)],
                            cnt.at[pl.ds(r0, RPT)])

            def fill_ones(i, carry):
                ones_v[i, :] = jnp.full((16,), 1.0, jnp.float32)
                return carry
            lax.fori_loop(0, _CH, fill_ones, 0)
        plsc.subcore_barrier()

        e0 = s * EPT

        def step(i, carry):
            off = e0 + i * _CH
            pltpu.sync_copy(ei_hbm.at[c, 0, pl.ds(off, _CH)], idx_s.at[0])
            pltpu.sync_copy(ei_hbm.at[c, 1, pl.ds(off, _CH)], idx_d.at[0])
            pltpu.async_copy(h_hbm.at[idx_s.at[0]], rows, sem).wait()
            pltpu.sync_copy(rows, acc.at[idx_d.at[0]], add=True)
            if with_counts:
                pltpu.sync_copy(ones_v, cnt.at[idx_d.at[0]], add=True)
            return carry
        lax.fori_loop(0, NCH, step, 0)

        plsc.subcore_barrier()
        pltpu.sync_copy(acc.at[pl.ds(r0, RPT)],
                        out_sums.at[c, pl.ds(r0, RPT)])
        if with_counts:
            pltpu.sync_copy(cnt.at[pl.ds(r0, RPT)],
                            out_cnts.at[c, pl.ds(r0, RPT)])

    return pl.kernel(body, out_type=tuple(out_type), mesh=mesh,
                     scratch_types=tuple(scratch),
                     compiler_params=pltpu.CompilerParams(
                         use_tc_tiling_on_sc=False))


def _pre_body(x_ref, lnw, lnb, W1, b1, W2, b2, o_ref):
    x = x_ref[...]
    m = jnp.mean(x, -1, keepdims=True)
    v = jnp.mean((x - m) ** 2, -1, keepdims=True)
    h = (x - m) / jnp.sqrt(v + _EPS) * lnw[...] + lnb[...]
    h = _gelu(jnp.dot(h, W1[...], preferred_element_type=jnp.float32) + b1[...])
    h = _gelu(jnp.dot(h, W2[...], preferred_element_type=jnp.float32) + b2[...])
    o_ref[...] = h


def _conv_body(h_ref, sums_ref, cnts_ref, Wl0, bl0, Wr0, Wl1, bl1, Wr1,
               t_ref, st_ref):
    h = h_ref[...]
    c0 = jnp.maximum(cnts_ref[0, :, 0:1], 1.0)
    c1 = jnp.maximum(cnts_ref[1, :, 0:1], 1.0)
    a0 = sums_ref[0] / c0
    a1 = sums_ref[1] / c1
    o0 = (jnp.dot(a0, Wl0[...], preferred_element_type=jnp.float32) + bl0[...]
          + jnp.dot(h, Wr0[...], preferred_element_type=jnp.float32))
    o1 = (jnp.dot(a1, Wl1[...], preferred_element_type=jnp.float32) + bl1[...]
          + jnp.dot(h, Wr1[...], preferred_element_type=jnp.float32))
    t = _gelu((o0 + o1) * 0.5)
    t_ref[...] = t
    part = jnp.concatenate([jnp.sum(t, 0)[None], jnp.sum(t * t, 0)[None]], 0)

    @pl.when(pl.program_id(0) == 0)
    def _():
        st_ref[...] = jnp.zeros_like(st_ref)
    st_ref[...] += part


def _gn_body(n_rows, t_ref, st_ref, w, b, alpha, o_ref):
    t = t_ref[...]
    m = st_ref[0:1, :] / n_rows
    q = st_ref[1:2, :] / n_rows
    a = alpha[...]
    var = q - 2.0 * a * m * m + a * a * m * m
    o_ref[...] = w[...] * (t - m * a) / jnp.sqrt(var + _EPS) + b[...]


def _head_body(kcode, n_out, h1_ref, h2_ref, aa_ref, hW1, hb1, hW2, hb2,
               oW1, ob1, oW2, ob2, oW3, ob3, cb, z_ref, loss_ref):
    x = jnp.concatenate([h1_ref[...], h2_ref[...]], 1)
    x = _gelu(jnp.dot(x, hW1[...], preferred_element_type=jnp.float32) + hb1[...])
    x = _gelu(jnp.dot(x, hW2[...], preferred_element_type=jnp.float32) + hb2[...])
    x = jnp.concatenate([x, aa_ref[...]], 1)
    x = _gelu(jnp.dot(x, oW1[...], preferred_element_type=jnp.float32) + ob1[...])
    x = _gelu(jnp.dot(x, oW2[...], preferred_element_type=jnp.float32) + ob2[...])
    x = jnp.tanh(jnp.dot(x, oW3[...], preferred_element_type=jnp.float32) + ob3[...])
    cbv = cb[...]
    d = (jnp.sum(x * x, 1, keepdims=True)
         - 2.0 * lax.dot_general(x, cbv, (((1,), (1,)), ((), ())),
                                 preferred_element_type=jnp.float32)
         + jnp.sum(cbv * cbv, 1)[None, :])
    dmin = jnp.min(d, 1, keepdims=True)
    iota = lax.broadcasted_iota(jnp.int32, d.shape, 1)
    amin = jnp.min(jnp.where(d == dmin, iota, kcode), 1, keepdims=True)
    onehot = (iota == amin).astype(jnp.float32)
    zq = jnp.dot(onehot, cbv, preferred_element_type=jnp.float32)
    z_ref[...] = x + (zq - x)
    p = jnp.sum((zq - x) ** 2)

    first = pl.program_id(0) == 0
    last = pl.program_id(0) == pl.num_programs(0) - 1

    @pl.when(first)
    def _():
        loss_ref[...] = jnp.zeros_like(loss_ref)
    loss_ref[...] += p

    @pl.when(last)
    def _():
        loss_ref[...] *= 1.25 / n_out


def _full(shape):
    return pl.BlockSpec(shape, lambda i: tuple(0 for _ in shape))


def kernel(x_res, x_AA, edge_index_backbone, edge_index_contact, params):
    p = params
    N, IN_C = x_res.shape
    HID = p['in_W2'].shape[1]
    E = edge_index_backbone.shape[1]
    KCODE, OUT_C = p['codebook'].shape
    AA = x_AA.shape[1]
    ENC_H = p['h_W1'].shape[1]
    nblk = N // _BLK

    r2 = lambda a: a.reshape(1, -1)

    h = pl.pallas_call(
        _pre_body,
        grid=(nblk,),
        in_specs=[
            pl.BlockSpec((_BLK, IN_C), lambda i: (i, 0)),
            _full((1, IN_C)), _full((1, IN_C)),
            _full(p['in_W1'].shape), _full((1, p['in_W1'].shape[1])),
            _full(p['in_W2'].shape), _full((1, HID)),
        ],
        out_specs=pl.BlockSpec((_BLK, HID), lambda i: (i, 0)),
        out_shape=jax.ShapeDtypeStruct((N, HID), jnp.float32),
    )(x_res, r2(p['ln_w']), r2(p['ln_b']),
      p['in_W1'], r2(p['in_b1']), p['in_W2'], r2(p['in_b2']))

    ei_all = jnp.stack([edge_index_backbone, edge_index_contact], 0)
    NPAD = ((N + 8 * _N_TILES - 1) // (8 * _N_TILES)) * (8 * _N_TILES)
    zeros = jnp.zeros((NPAD, HID), jnp.float32)

    sc_agg0 = _make_sc_agg(NPAD, HID, E, with_counts=True)
    sc_agg1 = _make_sc_agg(NPAD, HID, E, with_counts=False)

    conv_call = functools.partial(
        pl.pallas_call, _conv_body,
        grid=(nblk,),
        in_specs=[
            pl.BlockSpec((_BLK, HID), lambda i: (i, 0)),
            pl.BlockSpec((2, _BLK, HID), lambda i: (0, i, 0)),
            pl.BlockSpec((2, _BLK, 16), lambda i: (0, i, 0)),
            _full((HID, HID)), _full((1, HID)), _full((HID, HID)),
            _full((HID, HID)), _full((1, HID)), _full((HID, HID)),
        ],
        out_specs=[
            pl.BlockSpec((_BLK, HID), lambda i: (i, 0)),
            _full((2, HID)),
        ],
        out_shape=[
            jax.ShapeDtypeStruct((N, HID), jnp.float32),
            jax.ShapeDtypeStruct((2, HID), jnp.float32),
        ],
    )()

    gn_call = functools.partial(
        pl.pallas_call, functools.partial(_gn_body, float(N)),
        grid=(nblk,),
        in_specs=[
            pl.BlockSpec((_BLK, HID), lambda i: (i, 0)),
            _full((2, HID)),
            _full((1, HID)), _full((1, HID)), _full((1, HID)),
        ],
        out_specs=pl.BlockSpec((_BLK, HID), lambda i: (i, 0)),
        out_shape=jax.ShapeDtypeStruct((N, HID), jnp.float32),
    )()

    x_save = []
    cnts = None
    for i in range(2):
        if i == 0:
            sums, cnts = sc_agg0(h, ei_all, zeros)
        else:
            (sums,) = sc_agg1(h, ei_all, zeros)
        cv = p['convs'][i]
        t, st = conv_call(
            h, sums, cnts,
            cv[0]['Wl'], r2(cv[0]['bl']), cv[0]['Wr'],
            cv[1]['Wl'], r2(cv[1]['bl']), cv[1]['Wr'])
        g = p['gns'][i]
        h = gn_call(t, st, r2(g['w']), r2(g['b']), r2(g['alpha']))
        x_save.append(h)

    z, loss_v = pl.pallas_call(
        functools.partial(_head_body, KCODE, float(N * OUT_C)),
        grid=(nblk,),
        in_specs=[
            pl.BlockSpec((_BLK, HID), lambda i: (i, 0)),
            pl.BlockSpec((_BLK, HID), lambda i: (i, 0)),
            pl.BlockSpec((_BLK, AA), lambda i: (i, 0)),
            _full(p['h_W1'].shape), _full((1, ENC_H)),
            _full(p['h_W2'].shape), _full((1, ENC_H)),
            _full(p['o_W1'].shape), _full((1, ENC_H)),
            _full(p['o_W2'].shape), _full((1, ENC_H)),
            _full(p['o_W3'].shape), _full((1, OUT_C)),
            _full((KCODE, OUT_C)),
        ],
        out_specs=[
            pl.BlockSpec((_BLK, OUT_C), lambda i: (i, 0)),
            _full((1, 128)),
        ],
        out_shape=[
            jax.ShapeDtypeStruct((N, OUT_C), jnp.float32),
            jax.ShapeDtypeStruct((1, 128), jnp.float32),
        ],
    )(x_save[0], x_save[1], x_AA,
      p['h_W1'], r2(p['h_b1']), p['h_W2'], r2(p['h_b2']),
      p['o_W1'], r2(p['o_b1']), p['o_W2'], r2(p['o_b2']),
      p['o_W3'], r2(p['o_b3']), p['codebook'])

    return z, loss_v[0, 0]


# speedup vs baseline: 10.7796x; 2.6103x over previous
import functools

import jax
import jax.numpy as jnp
from jax import lax
from jax.experimental import pallas as pl
from jax.experimental.pallas import tpu as pltpu
from jax.experimental.pallas import tpu_sc as plsc

_N_TILES = 16
_CH = 80
_NB = 4
_BLK = 2000
_EPS = 1e-5


def _gelu(x):
    return 0.5 * x * (1.0 + lax.erf(x * (2.0 ** -0.5)))


def _make_sc_agg(NPAD, F, E):
    EPT = E // _N_TILES
    NCH = EPT // _CH
    RPT = NPAD // _N_TILES
    mesh = plsc.VectorSubcoreMesh(core_axis_name="c", subcore_axis_name="s")

    NI = 2 * _NB
    out_type = jax.ShapeDtypeStruct((2, NPAD, F), jnp.float32)
    scratch = (
        [pltpu.VMEM((1, _CH), jnp.int32) for _ in range(2 * NI)]
        + [pltpu.VMEM((_CH, F), jnp.float32) for _ in range(_NB)]
        + [pltpu.SemaphoreType.DMA for _ in range(NI)]
        + [pltpu.SemaphoreType.DMA for _ in range(2 * _NB)]
        + [pltpu.VMEM_SHARED((NPAD, F), jnp.float32)]
    )

    def body(h_hbm, ei_hbm, z_hbm, out_sums, *rest):
        idx_s = rest[0:NI]
        idx_d = rest[NI:2 * NI]
        rows = rest[2 * NI:2 * NI + _NB]
        isem = rest[2 * NI + _NB:3 * NI + _NB]
        gsem = rest[3 * NI + _NB:3 * NI + 2 * _NB]
        ssem = rest[3 * NI + 2 * _NB:3 * NI + 3 * _NB]
        acc = rest[3 * NI + 3 * _NB]

        c = lax.axis_index("c")
        s = lax.axis_index("s")
        r0 = s * RPT
        pltpu.sync_copy(z_hbm.at[pl.ds(r0, RPT)], acc.at[pl.ds(r0, RPT)])
        plsc.subcore_barrier()

        j0 = s * NCH
        last = NCH - 1

        def i_fire(t, p):
            t = jnp.minimum(t, last) + j0
            pltpu.async_copy(ei_hbm.at[c, 0, pl.ds(t, 1)], idx_s[p], isem[p])
            pltpu.async_copy(ei_hbm.at[c, 1, pl.ds(t, 1)], idx_d[p], isem[p])

        def i_wait(t, p):
            t = jnp.minimum(t, last) + j0
            pltpu.make_async_copy(ei_hbm.at[c, 0, pl.ds(t, 1)], idx_s[p],
                                  isem[p]).wait()
            pltpu.make_async_copy(ei_hbm.at[c, 1, pl.ds(t, 1)], idx_d[p],
                                  isem[p]).wait()

        def g_fire(t, p, b):
            pltpu.async_copy(h_hbm.at[idx_s[p].at[0]], rows[b], gsem[b])

        def g_wait(t, p, b):
            pltpu.make_async_copy(h_hbm.at[idx_s[p].at[0]], rows[b],
                                  gsem[b]).wait()

        def s_fire(t, p, b):
            pltpu.async_copy(rows[b], acc.at[idx_d[p].at[0]], ssem[b],
                             add=True)

        def s_wait(t, p, b):
            pltpu.make_async_copy(rows[b], acc.at[idx_d[p].at[0]],
                                  ssem[b]).wait()

        def full_step(t, u):
            q, q2, p, p4 = u % _NB, (u + 2) % _NB, u % NI, (u + _NB) % NI
            s_wait(t - _NB, (u - _NB) % NI, q)
            i_fire(t + _NB, p4)
            i_wait(t, p)
            g_fire(t, p, q)
            g_wait(t - 2, (u - 2) % NI, q2)
            s_fire(t - 2, (u - 2) % NI, q2)

        for u in range(_NB):
            i_fire(u, u % NI)
        for u in range(_NB):
            i_wait(u, u % NI)
            g_fire(u, u % NI, u % _NB)
            i_fire(u + _NB, (u + _NB) % NI)
            if u >= 2:
                g_wait(u - 2, (u - 2) % NI, (u - 2) % _NB)
                s_fire(u - 2, (u - 2) % NI, (u - 2) % _NB)
        for u in range(_NB, NI):
            full_step(u, u)

        def main(T, carry):
            base = NI + T * NI
            for u in range(NI):
                full_step(base + u, u)
            return carry
        nmain = (NCH - NI) // NI
        lax.fori_loop(0, nmain, main, 0)

        for t in range(NCH - _NB, NI + nmain * NI):
            i_wait(last, (t % NI + _NB) % NI)
        for t in range(NI + nmain * NI, NCH):
            u = t % NI
            s_wait(t - _NB, (u - _NB) % NI, u % _NB)
            i_wait(t, u)
            g_fire(t, u, u % _NB)
            g_wait(t - 2, (u - 2) % NI, (u - 2) % _NB)
            s_fire(t - 2, (u - 2) % NI, (u - 2) % _NB)
        for t in range(NCH - 2, NCH):
            u = t % NI
            g_wait(t, u, t % _NB)
            s_fire(t, u, t % _NB)
        for t in range(NCH - _NB, NCH):
            s_wait(t, t % NI, t % _NB)

        plsc.subcore_barrier()
        pltpu.sync_copy(acc.at[pl.ds(r0, RPT)],
                        out_sums.at[c, pl.ds(r0, RPT)])

    return pl.kernel(body, out_type=out_type, mesh=mesh,
                     scratch_types=tuple(scratch),
                     compiler_params=pltpu.CompilerParams(
                         use_tc_tiling_on_sc=False))


def _make_sc_counts(N, E):
    EPT = E // _N_TILES
    NCH = EPT // _CH
    RPTC = N // _N_TILES
    NI = 2 * _NB
    mesh = plsc.VectorSubcoreMesh(core_axis_name="c", subcore_axis_name="s")
    out_type = jax.ShapeDtypeStruct((2, N, 16), jnp.float32)
    scratch = (
        [pltpu.VMEM((1, _CH), jnp.int32) for _ in range(NI)]
        + [pltpu.SemaphoreType.DMA for _ in range(NI)]
        + [pltpu.SemaphoreType.DMA for _ in range(NI)]
        + [pltpu.VMEM((_CH, 16), jnp.float32)]
        + [pltpu.VMEM_SHARED((N, 16), jnp.float32)]
    )

    def body(ei_hbm, z_hbm, one_hbm, out_cnts, *rest):
        idx_d = rest[0:NI]
        isem = rest[NI:2 * NI]
        csem = rest[2 * NI:3 * NI]
        ones_v = rest[3 * NI]
        cnt = rest[3 * NI + 1]

        c = lax.axis_index("c")
        s = lax.axis_index("s")
        r0c = s * RPTC
        pltpu.sync_copy(z_hbm.at[pl.ds(r0c, RPTC), pl.ds(0, 16)],
                        cnt.at[pl.ds(r0c, RPTC)])
        pltpu.sync_copy(one_hbm, ones_v)
        plsc.subcore_barrier()

        j0 = s * NCH
        last = NCH - 1

        def i_fire(t, p):
            t = jnp.minimum(t, last) + j0
            pltpu.async_copy(ei_hbm.at[c, 1, pl.ds(t, 1)], idx_d[p], isem[p])

        def i_wait(t, p):
            t = jnp.minimum(t, last) + j0
            pltpu.make_async_copy(ei_hbm.at[c, 1, pl.ds(t, 1)], idx_d[p],
                                  isem[p]).wait()

        def c_fire(t, p):
            pltpu.async_copy(ones_v, cnt.at[idx_d[p].at[0]], csem[p],
                             add=True)

        def c_wait(t, p):
            pltpu.make_async_copy(ones_v, cnt.at[idx_d[p].at[0]],
                                  csem[p]).wait()

        def full_step(t, u):
            p, p4 = u % NI, (u + _NB) % NI
            c_wait(t - _NB, (u - _NB) % NI)
            i_fire(t + _NB, p4)
            i_wait(t, p)
            c_fire(t, p)

        for u in range(_NB):
            i_fire(u, u)
        for u in range(_NB):
            i_wait(u, u)
            c_fire(u, u)
            i_fire(u + _NB, u + _NB)
        for u in range(_NB, NI):
            full_step(u, u)

        def main(T, carry):
            base = NI + T * NI
            for u in range(NI):
                full_step(base + u, u)
            return carry
        nmain = (NCH - NI) // NI
        lax.fori_loop(0, nmain, main, 0)

        for t in range(NCH - _NB, NI + nmain * NI):
            i_wait(last, (t % NI + _NB) % NI)
        for t in range(NI + nmain * NI, NCH):
            u = t % NI
            c_wait(t - _NB, (u - _NB) % NI)
            i_wait(t, u)
            c_fire(t, u)
        for t in range(NCH - _NB, NCH):
            c_wait(t, t % NI)

        plsc.subcore_barrier()
        pltpu.sync_copy(cnt.at[pl.ds(r0c, RPTC)],
                        out_cnts.at[c, pl.ds(r0c, RPTC)])

    return pl.kernel(body, out_type=out_type, mesh=mesh,
                     scratch_types=tuple(scratch),
                     compiler_params=pltpu.CompilerParams(
                         use_tc_tiling_on_sc=False))


def _pre_body(x_ref, lnw, lnb, W1, b1, W2, b2, o_ref):
    x = x_ref[...]
    m = jnp.mean(x, -1, keepdims=True)
    v = jnp.mean((x - m) ** 2, -1, keepdims=True)
    h = (x - m) / jnp.sqrt(v + _EPS) * lnw[...] + lnb[...]
    h = _gelu(jnp.dot(h, W1[...], preferred_element_type=jnp.float32) + b1[...])
    h = _gelu(jnp.dot(h, W2[...], preferred_element_type=jnp.float32) + b2[...])
    o_ref[...] = h


def _conv_body(h_ref, sums_ref, cnts_ref, Wl0, bl0, Wr0, Wl1, bl1, Wr1,
               t_ref, st_ref):
    h = h_ref[...]
    c0 = jnp.maximum(cnts_ref[0, :, 0:1], 1.0)
    c1 = jnp.maximum(cnts_ref[1, :, 0:1], 1.0)
    a0 = sums_ref[0] / c0
    a1 = sums_ref[1] / c1
    o0 = (jnp.dot(a0, Wl0[...], preferred_element_type=jnp.float32) + bl0[...]
          + jnp.dot(h, Wr0[...], preferred_element_type=jnp.float32))
    o1 = (jnp.dot(a1, Wl1[...], preferred_element_type=jnp.float32) + bl1[...]
          + jnp.dot(h, Wr1[...], preferred_element_type=jnp.float32))
    t = _gelu((o0 + o1) * 0.5)
    t_ref[...] = t
    part = jnp.concatenate([jnp.sum(t, 0)[None], jnp.sum(t * t, 0)[None]], 0)

    @pl.when(pl.program_id(0) == 0)
    def _():
        st_ref[...] = jnp.zeros_like(st_ref)
    st_ref[...] += part


def _gn_body(n_rows, t_ref, st_ref, w, b, alpha, o_ref):
    t = t_ref[...]
    m = st_ref[0:1, :] / n_rows
    q = st_ref[1:2, :] / n_rows
    a = alpha[...]
    var = q - 2.0 * a * m * m + a * a * m * m
    o_ref[...] = w[...] * (t - m * a) / jnp.sqrt(var + _EPS) + b[...]


def _head_body(kcode, n_out, h1_ref, h2_ref, aa_ref, hW1, hb1, hW2, hb2,
               oW1, ob1, oW2, ob2, oW3, ob3, cb, z_ref, loss_ref):
    x = jnp.concatenate([h1_ref[...], h2_ref[...]], 1)
    x = _gelu(jnp.dot(x, hW1[...], preferred_element_type=jnp.float32) + hb1[...])
    x = _gelu(jnp.dot(x, hW2[...], preferred_element_type=jnp.float32) + hb2[...])
    x = jnp.concatenate([x, aa_ref[...]], 1)
    x = _gelu(jnp.dot(x, oW1[...], preferred_element_type=jnp.float32) + ob1[...])
    x = _gelu(jnp.dot(x, oW2[...], preferred_element_type=jnp.float32) + ob2[...])
    x = jnp.tanh(jnp.dot(x, oW3[...], preferred_element_type=jnp.float32) + ob3[...])
    cbv = cb[...]
    d = (jnp.sum(x * x, 1, keepdims=True)
         - 2.0 * lax.dot_general(x, cbv, (((1,), (1,)), ((), ())),
                                 preferred_element_type=jnp.float32)
         + jnp.sum(cbv * cbv, 1)[None, :])
    dmin = jnp.min(d, 1, keepdims=True)
    iota = lax.broadcasted_iota(jnp.int32, d.shape, 1)
    amin = jnp.min(jnp.where(d == dmin, iota, kcode), 1, keepdims=True)
    onehot = (iota == amin).astype(jnp.float32)
    zq = jnp.dot(onehot, cbv, preferred_element_type=jnp.float32)
    z_ref[...] = x + (zq - x)
    p = jnp.sum((zq - x) ** 2)

    first = pl.program_id(0) == 0
    last = pl.program_id(0) == pl.num_programs(0) - 1

    @pl.when(first)
    def _():
        loss_ref[...] = jnp.zeros_like(loss_ref)
    loss_ref[...] += p

    @pl.when(last)
    def _():
        loss_ref[...] *= 1.25 / n_out


def _full(shape):
    return pl.BlockSpec(shape, lambda i: tuple(0 for _ in shape))


def kernel(x_res, x_AA, edge_index_backbone, edge_index_contact, params):
    p = params
    N, IN_C = x_res.shape
    HID = p['in_W2'].shape[1]
    E = edge_index_backbone.shape[1]
    KCODE, OUT_C = p['codebook'].shape
    AA = x_AA.shape[1]
    ENC_H = p['h_W1'].shape[1]
    nblk = N // _BLK

    r2 = lambda a: a.reshape(1, -1)

    h = pl.pallas_call(
        _pre_body,
        grid=(nblk,),
        in_specs=[
            pl.BlockSpec((_BLK, IN_C), lambda i: (i, 0)),
            _full((1, IN_C)), _full((1, IN_C)),
            _full(p['in_W1'].shape), _full((1, p['in_W1'].shape[1])),
            _full(p['in_W2'].shape), _full((1, HID)),
        ],
        out_specs=pl.BlockSpec((_BLK, HID), lambda i: (i, 0)),
        out_shape=jax.ShapeDtypeStruct((N, HID), jnp.float32),
    )(x_res, r2(p['ln_w']), r2(p['ln_b']),
      p['in_W1'], r2(p['in_b1']), p['in_W2'], r2(p['in_b2']))

    ei_all = jnp.stack([edge_index_backbone, edge_index_contact], 0)
    ei_all = ei_all.reshape(2, 2, E // _CH, _CH)
    NPAD = ((N + 8 * _N_TILES - 1) // (8 * _N_TILES)) * (8 * _N_TILES)
    zeros = jnp.zeros((NPAD, HID), jnp.float32)
    ones = jnp.ones((_CH, 16), jnp.float32)

    sc_agg = _make_sc_agg(NPAD, HID, E)
    sc_counts = _make_sc_counts(N, E)
    cnts = sc_counts(ei_all, zeros, ones)

    conv_call = functools.partial(
        pl.pallas_call, _conv_body,
        grid=(nblk,),
        in_specs=[
            pl.BlockSpec((_BLK, HID), lambda i: (i, 0)),
            pl.BlockSpec((2, _BLK, HID), lambda i: (0, i, 0)),
            pl.BlockSpec((2, _BLK, 16), lambda i: (0, i, 0)),
            _full((HID, HID)), _full((1, HID)), _full((HID, HID)),
            _full((HID, HID)), _full((1, HID)), _full((HID, HID)),
        ],
        out_specs=[
            pl.BlockSpec((_BLK, HID), lambda i: (i, 0)),
            _full((2, HID)),
        ],
        out_shape=[
            jax.ShapeDtypeStruct((N, HID), jnp.float32),
            jax.ShapeDtypeStruct((2, HID), jnp.float32),
        ],
    )()

    gn_call = functools.partial(
        pl.pallas_call, functools.partial(_gn_body, float(N)),
        grid=(nblk,),
        in_specs=[
            pl.BlockSpec((_BLK, HID), lambda i: (i, 0)),
            _full((2, HID)),
            _full((1, HID)), _full((1, HID)), _full((1, HID)),
        ],
        out_specs=pl.BlockSpec((_BLK, HID), lambda i: (i, 0)),
        out_shape=jax.ShapeDtypeStruct((N, HID), jnp.float32),
    )()

    x_save = []
    for i in range(2):
        sums = sc_agg(h, ei_all, zeros)
        cv = p['convs'][i]
        t, st = conv_call(
            h, sums, cnts,
            cv[0]['Wl'], r2(cv[0]['bl']), cv[0]['Wr'],
            cv[1]['Wl'], r2(cv[1]['bl']), cv[1]['Wr'])
        g = p['gns'][i]
        h = gn_call(t, st, r2(g['w']), r2(g['b']), r2(g['alpha']))
        x_save.append(h)

    z, loss_v = pl.pallas_call(
        functools.partial(_head_body, KCODE, float(N * OUT_C)),
        grid=(nblk,),
        in_specs=[
            pl.BlockSpec((_BLK, HID), lambda i: (i, 0)),
            pl.BlockSpec((_BLK, HID), lambda i: (i, 0)),
            pl.BlockSpec((_BLK, AA), lambda i: (i, 0)),
            _full(p['h_W1'].shape), _full((1, ENC_H)),
            _full(p['h_W2'].shape), _full((1, ENC_H)),
            _full(p['o_W1'].shape), _full((1, ENC_H)),
            _full(p['o_W2'].shape), _full((1, ENC_H)),
            _full(p['o_W3'].shape), _full((1, OUT_C)),
            _full((KCODE, OUT_C)),
        ],
        out_specs=[
            pl.BlockSpec((_BLK, OUT_C), lambda i: (i, 0)),
            _full((1, 128)),
        ],
        out_shape=[
            jax.ShapeDtypeStruct((N, OUT_C), jnp.float32),
            jax.ShapeDtypeStruct((1, 128), jnp.float32),
        ],
    )(x_save[0], x_save[1], x_AA,
      p['h_W1'], r2(p['h_b1']), p['h_W2'], r2(p['h_b2']),
      p['o_W1'], r2(p['o_b1']), p['o_W2'], r2(p['o_b2']),
      p['o_W3'], r2(p['o_b3']), p['codebook'])

    return z, loss_v[0, 0]


# speedup vs baseline: 11.1213x; 1.0317x over previous
import functools

import jax
import jax.numpy as jnp
from jax import lax
from jax.experimental import pallas as pl
from jax.experimental.pallas import tpu as pltpu
from jax.experimental.pallas import tpu_sc as plsc

_N_TILES = 16
_CH = 80
_NB = 4
_BLK = 2000
_EPS = 1e-5


def _gelu(x):
    return 0.5 * x * (1.0 + lax.erf(x * (2.0 ** -0.5)))


def _make_sc_agg(NPAD, F, E):
    EPT = E // _N_TILES
    NCH = EPT // _CH
    RPT = NPAD // _N_TILES
    mesh = plsc.VectorSubcoreMesh(core_axis_name="c", subcore_axis_name="s")

    NI = 2 * _NB
    out_type = jax.ShapeDtypeStruct((2, NPAD, F), jnp.float32)
    scratch = (
        [pltpu.VMEM((_CH,), jnp.int32) for _ in range(2 * NI)]
        + [pltpu.VMEM((_CH, F), jnp.float32) for _ in range(_NB)]
        + [pltpu.SemaphoreType.DMA for _ in range(NI)]
        + [pltpu.SemaphoreType.DMA for _ in range(2 * _NB)]
        + [pltpu.VMEM_SHARED((NPAD, F), jnp.float32)]
    )

    def body(h_hbm, eb_hbm, ec_hbm, z_hbm, out_sums, *rest):
        idx_s = rest[0:NI]
        idx_d = rest[NI:2 * NI]
        rows = rest[2 * NI:2 * NI + _NB]
        isem = rest[2 * NI + _NB:3 * NI + _NB]
        gsem = rest[3 * NI + _NB:3 * NI + 2 * _NB]
        ssem = rest[3 * NI + 2 * _NB:3 * NI + 3 * _NB]
        acc = rest[3 * NI + 3 * _NB]

        c = lax.axis_index("c")
        s = lax.axis_index("s")
        r0 = s * RPT
        pltpu.sync_copy(z_hbm.at[pl.ds(r0, RPT)], acc.at[pl.ds(r0, RPT)])
        plsc.subcore_barrier()

        j0 = s * NCH
        last = NCH - 1

        def i_fire(t, p):
            off = (jnp.minimum(t, last) + j0) * _CH

            @pl.when(c == 0)
            def _():
                pltpu.async_copy(eb_hbm.at[0, pl.ds(off, _CH)], idx_s[p],
                                 isem[p])
                pltpu.async_copy(eb_hbm.at[1, pl.ds(off, _CH)], idx_d[p],
                                 isem[p])

            @pl.when(c != 0)
            def _():
                pltpu.async_copy(ec_hbm.at[0, pl.ds(off, _CH)], idx_s[p],
                                 isem[p])
                pltpu.async_copy(ec_hbm.at[1, pl.ds(off, _CH)], idx_d[p],
                                 isem[p])

        def i_wait(t, p):
            off = (jnp.minimum(t, last) + j0) * _CH
            pltpu.make_async_copy(eb_hbm.at[0, pl.ds(off, _CH)], idx_s[p],
                                  isem[p]).wait()
            pltpu.make_async_copy(eb_hbm.at[1, pl.ds(off, _CH)], idx_d[p],
                                  isem[p]).wait()

        def g_fire(t, p, b):
            pltpu.async_copy(h_hbm.at[idx_s[p]], rows[b], gsem[b])

        def g_wait(t, p, b):
            pltpu.make_async_copy(h_hbm.at[idx_s[p]], rows[b],
                                  gsem[b]).wait()

        def s_fire(t, p, b):
            pltpu.async_copy(rows[b], acc.at[idx_d[p]], ssem[b],
                             add=True)

        def s_wait(t, p, b):
            pltpu.make_async_copy(rows[b], acc.at[idx_d[p]],
                                  ssem[b]).wait()

        def full_step(t, u):
            q, q2, p, p4 = u % _NB, (u + 2) % _NB, u % NI, (u + _NB) % NI
            s_wait(t - _NB, (u - _NB) % NI, q)
            i_fire(t + _NB, p4)
            i_wait(t, p)
            g_fire(t, p, q)
            g_wait(t - 2, (u - 2) % NI, q2)
            s_fire(t - 2, (u - 2) % NI, q2)

        for u in range(_NB):
            i_fire(u, u % NI)
        for u in range(_NB):
            i_wait(u, u % NI)
            g_fire(u, u % NI, u % _NB)
            i_fire(u + _NB, (u + _NB) % NI)
            if u >= 2:
                g_wait(u - 2, (u - 2) % NI, (u - 2) % _NB)
                s_fire(u - 2, (u - 2) % NI, (u - 2) % _NB)
        for u in range(_NB, NI):
            full_step(u, u)

        def main(T, carry):
            base = NI + T * NI
            for u in range(NI):
                full_step(base + u, u)
            return carry
        nmain = (NCH - NI) // NI
        lax.fori_loop(0, nmain, main, 0)

        for t in range(NCH - _NB, NI + nmain * NI):
            i_wait(last, (t % NI + _NB) % NI)
        for t in range(NI + nmain * NI, NCH):
            u = t % NI
            s_wait(t - _NB, (u - _NB) % NI, u % _NB)
            i_wait(t, u)
            g_fire(t, u, u % _NB)
            g_wait(t - 2, (u - 2) % NI, (u - 2) % _NB)
            s_fire(t - 2, (u - 2) % NI, (u - 2) % _NB)
        for t in range(NCH - 2, NCH):
            u = t % NI
            g_wait(t, u, t % _NB)
            s_fire(t, u, t % _NB)
        for t in range(NCH - _NB, NCH):
            s_wait(t, t % NI, t % _NB)

        plsc.subcore_barrier()
        pltpu.sync_copy(acc.at[pl.ds(r0, RPT)],
                        out_sums.at[c, pl.ds(r0, RPT)])

    return pl.kernel(body, out_type=out_type, mesh=mesh,
                     scratch_types=tuple(scratch),
                     compiler_params=pltpu.CompilerParams(
                         use_tc_tiling_on_sc=False))


def _make_sc_counts(N, E):
    EPT = E // _N_TILES
    NCH = EPT // _CH
    RPTC = N // _N_TILES
    NI = 2 * _NB
    mesh = plsc.VectorSubcoreMesh(core_axis_name="c", subcore_axis_name="s")
    out_type = jax.ShapeDtypeStruct((2, N, 16), jnp.float32)
    scratch = (
        [pltpu.VMEM((_CH,), jnp.int32) for _ in range(NI)]
        + [pltpu.SemaphoreType.DMA for _ in range(NI)]
        + [pltpu.SemaphoreType.DMA for _ in range(NI)]
        + [pltpu.VMEM((_CH, 16), jnp.float32)]
        + [pltpu.VMEM_SHARED((N, 16), jnp.float32)]
    )

    def body(eb_hbm, ec_hbm, z_hbm, one_hbm, out_cnts, *rest):
        idx_d = rest[0:NI]
        isem = rest[NI:2 * NI]
        csem = rest[2 * NI:3 * NI]
        ones_v = rest[3 * NI]
        cnt = rest[3 * NI + 1]

        c = lax.axis_index("c")
        s = lax.axis_index("s")
        r0c = s * RPTC
        pltpu.sync_copy(z_hbm.at[pl.ds(r0c, RPTC), pl.ds(0, 16)],
                        cnt.at[pl.ds(r0c, RPTC)])
        pltpu.sync_copy(one_hbm, ones_v)
        plsc.subcore_barrier()

        j0 = s * NCH
        last = NCH - 1

        def i_fire(t, p):
            off = (jnp.minimum(t, last) + j0) * _CH

            @pl.when(c == 0)
            def _():
                pltpu.async_copy(eb_hbm.at[1, pl.ds(off, _CH)], idx_d[p],
                                 isem[p])

            @pl.when(c != 0)
            def _():
                pltpu.async_copy(ec_hbm.at[1, pl.ds(off, _CH)], idx_d[p],
                                 isem[p])

        def i_wait(t, p):
            off = (jnp.minimum(t, last) + j0) * _CH
            pltpu.make_async_copy(eb_hbm.at[1, pl.ds(off, _CH)], idx_d[p],
                                  isem[p]).wait()

        def c_fire(t, p):
            pltpu.async_copy(ones_v, cnt.at[idx_d[p]], csem[p],
                             add=True)

        def c_wait(t, p):
            pltpu.make_async_copy(ones_v, cnt.at[idx_d[p]],
                                  csem[p]).wait()

        def full_step(t, u):
            p, p4 = u % NI, (u + _NB) % NI
            c_wait(t - _NB, (u - _NB) % NI)
            i_fire(t + _NB, p4)
            i_wait(t, p)
            c_fire(t, p)

        for u in range(_NB):
            i_fire(u, u)
        for u in range(_NB):
            i_wait(u, u)
            c_fire(u, u)
            i_fire(u + _NB, u + _NB)
        for u in range(_NB, NI):
            full_step(u, u)

        def main(T, carry):
            base = NI + T * NI
            for u in range(NI):
                full_step(base + u, u)
            return carry
        nmain = (NCH - NI) // NI
        lax.fori_loop(0, nmain, main, 0)

        for t in range(NCH - _NB, NI + nmain * NI):
            i_wait(last, (t % NI + _NB) % NI)
        for t in range(NI + nmain * NI, NCH):
            u = t % NI
            c_wait(t - _NB, (u - _NB) % NI)
            i_wait(t, u)
            c_fire(t, u)
        for t in range(NCH - _NB, NCH):
            c_wait(t, t % NI)

        plsc.subcore_barrier()
        pltpu.sync_copy(cnt.at[pl.ds(r0c, RPTC)],
                        out_cnts.at[c, pl.ds(r0c, RPTC)])

    return pl.kernel(body, out_type=out_type, mesh=mesh,
                     scratch_types=tuple(scratch),
                     compiler_params=pltpu.CompilerParams(
                         use_tc_tiling_on_sc=False))


def _pre_body(x_ref, lnw, lnb, W1, b1, W2, b2, o_ref):
    x = x_ref[...]
    m = jnp.mean(x, -1, keepdims=True)
    v = jnp.mean((x - m) ** 2, -1, keepdims=True)
    h = (x - m) / jnp.sqrt(v + _EPS) * lnw[...] + lnb[...]
    h = _gelu(jnp.dot(h, W1[...], preferred_element_type=jnp.float32) + b1[...])
    h = _gelu(jnp.dot(h, W2[...], preferred_element_type=jnp.float32) + b2[...])
    o_ref[...] = h


def _hw_body(h_ref, Wr0, Wr1, o_ref):
    o_ref[...] = jnp.dot(h_ref[...], Wr0[...] + Wr1[...],
                         preferred_element_type=jnp.float32)


def _conv_body(hw_ref, sums_ref, cnts_ref, Wl0, bl0, Wl1, bl1,
               t_ref, st_ref):
    c0 = jnp.maximum(cnts_ref[0, :, 0:1], 1.0)
    c1 = jnp.maximum(cnts_ref[1, :, 0:1], 1.0)
    a0 = sums_ref[0] / c0
    a1 = sums_ref[1] / c1
    o = (jnp.dot(a0, Wl0[...], preferred_element_type=jnp.float32) + bl0[...]
         + jnp.dot(a1, Wl1[...], preferred_element_type=jnp.float32)
         + bl1[...] + hw_ref[...])
    t = _gelu(o * 0.5)
    t_ref[...] = t
    part = jnp.concatenate([jnp.sum(t, 0)[None], jnp.sum(t * t, 0)[None]], 0)

    @pl.when(pl.program_id(0) == 0)
    def _():
        st_ref[...] = jnp.zeros_like(st_ref)
    st_ref[...] += part


def _gn_body(n_rows, t_ref, st_ref, w, b, alpha, o_ref):
    t = t_ref[...]
    m = st_ref[0:1, :] / n_rows
    q = st_ref[1:2, :] / n_rows
    a = alpha[...]
    var = q - 2.0 * a * m * m + a * a * m * m
    o_ref[...] = w[...] * (t - m * a) / jnp.sqrt(var + _EPS) + b[...]


def _head_body(kcode, n_out, h1_ref, h2_ref, aa_ref, hW1, hb1, hW2, hb2,
               oW1, ob1, oW2, ob2, oW3, ob3, cb, z_ref, loss_ref):
    x = jnp.concatenate([h1_ref[...], h2_ref[...]], 1)
    x = _gelu(jnp.dot(x, hW1[...], preferred_element_type=jnp.float32) + hb1[...])
    x = _gelu(jnp.dot(x, hW2[...], preferred_element_type=jnp.float32) + hb2[...])
    x = jnp.concatenate([x, aa_ref[...]], 1)
    x = _gelu(jnp.dot(x, oW1[...], preferred_element_type=jnp.float32) + ob1[...])
    x = _gelu(jnp.dot(x, oW2[...], preferred_element_type=jnp.float32) + ob2[...])
    x = jnp.tanh(jnp.dot(x, oW3[...], preferred_element_type=jnp.float32) + ob3[...])
    cbv = cb[...]
    d = (jnp.sum(x * x, 1, keepdims=True)
         - 2.0 * lax.dot_general(x, cbv, (((1,), (1,)), ((), ())),
                                 preferred_element_type=jnp.float32)
         + jnp.sum(cbv * cbv, 1)[None, :])
    dmin = jnp.min(d, 1, keepdims=True)
    iota = lax.broadcasted_iota(jnp.int32, d.shape, 1)
    amin = jnp.min(jnp.where(d == dmin, iota, kcode), 1, keepdims=True)
    onehot = (iota == amin).astype(jnp.float32)
    zq = jnp.dot(onehot, cbv, preferred_element_type=jnp.float32)
    z_ref[...] = x + (zq - x)
    p = jnp.sum((zq - x) ** 2)

    first = pl.program_id(0) == 0
    last = pl.program_id(0) == pl.num_programs(0) - 1

    @pl.when(first)
    def _():
        loss_ref[...] = jnp.zeros_like(loss_ref)
    loss_ref[...] += p

    @pl.when(last)
    def _():
        loss_ref[...] *= 1.25 / n_out


def _full(shape):
    return pl.BlockSpec(shape, lambda i: tuple(0 for _ in shape))


def kernel(x_res, x_AA, edge_index_backbone, edge_index_contact, params):
    p = params
    N, IN_C = x_res.shape
    HID = p['in_W2'].shape[1]
    E = edge_index_backbone.shape[1]
    KCODE, OUT_C = p['codebook'].shape
    AA = x_AA.shape[1]
    ENC_H = p['h_W1'].shape[1]
    nblk = N // _BLK

    r2 = lambda a: a.reshape(1, -1)

    h = pl.pallas_call(
        _pre_body,
        grid=(nblk,),
        in_specs=[
            pl.BlockSpec((_BLK, IN_C), lambda i: (i, 0)),
            _full((1, IN_C)), _full((1, IN_C)),
            _full(p['in_W1'].shape), _full((1, p['in_W1'].shape[1])),
            _full(p['in_W2'].shape), _full((1, HID)),
        ],
        out_specs=pl.BlockSpec((_BLK, HID), lambda i: (i, 0)),
        out_shape=jax.ShapeDtypeStruct((N, HID), jnp.float32),
    )(x_res, r2(p['ln_w']), r2(p['ln_b']),
      p['in_W1'], r2(p['in_b1']), p['in_W2'], r2(p['in_b2']))

    eb, ec = edge_index_backbone, edge_index_contact
    NPAD = ((N + 8 * _N_TILES - 1) // (8 * _N_TILES)) * (8 * _N_TILES)
    zeros = jnp.zeros((NPAD, HID), jnp.float32)
    ones = jnp.ones((_CH, 16), jnp.float32)

    sc_agg = _make_sc_agg(NPAD, HID, E)
    sc_counts = _make_sc_counts(N, E)
    cnts = sc_counts(eb, ec, zeros, ones)

    hw_call = functools.partial(
        pl.pallas_call, _hw_body,
        grid=(nblk,),
        in_specs=[
            pl.BlockSpec((_BLK, HID), lambda i: (i, 0)),
            _full((HID, HID)), _full((HID, HID)),
        ],
        out_specs=pl.BlockSpec((_BLK, HID), lambda i: (i, 0)),
        out_shape=jax.ShapeDtypeStruct((N, HID), jnp.float32),
    )()

    conv_call = functools.partial(
        pl.pallas_call, _conv_body,
        grid=(nblk,),
        in_specs=[
            pl.BlockSpec((_BLK, HID), lambda i: (i, 0)),
            pl.BlockSpec((2, _BLK, HID), lambda i: (0, i, 0)),
            pl.BlockSpec((2, _BLK, 16), lambda i: (0, i, 0)),
            _full((HID, HID)), _full((1, HID)),
            _full((HID, HID)), _full((1, HID)),
        ],
        out_specs=[
            pl.BlockSpec((_BLK, HID), lambda i: (i, 0)),
            _full((2, HID)),
        ],
        out_shape=[
            jax.ShapeDtypeStruct((N, HID), jnp.float32),
            jax.ShapeDtypeStruct((2, HID), jnp.float32),
        ],
    )()

    gn_call = functools.partial(
        pl.pallas_call, functools.partial(_gn_body, float(N)),
        grid=(nblk,),
        in_specs=[
            pl.BlockSpec((_BLK, HID), lambda i: (i, 0)),
            _full((2, HID)),
            _full((1, HID)), _full((1, HID)), _full((1, HID)),
        ],
        out_specs=pl.BlockSpec((_BLK, HID), lambda i: (i, 0)),
        out_shape=jax.ShapeDtypeStruct((N, HID), jnp.float32),
    )()

    x_save = []
    for i in range(2):
        sums = sc_agg(h, eb, ec, zeros)
        cv = p['convs'][i]
        hw = hw_call(h, cv[0]['Wr'], cv[1]['Wr'])
        t, st = conv_call(
            hw, sums, cnts,
            cv[0]['Wl'], r2(cv[0]['bl']),
            cv[1]['Wl'], r2(cv[1]['bl']))
        g = p['gns'][i]
        h = gn_call(t, st, r2(g['w']), r2(g['b']), r2(g['alpha']))
        x_save.append(h)

    z, loss_v = pl.pallas_call(
        functools.partial(_head_body, KCODE, float(N * OUT_C)),
        grid=(nblk,),
        in_specs=[
            pl.BlockSpec((_BLK, HID), lambda i: (i, 0)),
            pl.BlockSpec((_BLK, HID), lambda i: (i, 0)),
            pl.BlockSpec((_BLK, AA), lambda i: (i, 0)),
            _full(p['h_W1'].shape), _full((1, ENC_H)),
            _full(p['h_W2'].shape), _full((1, ENC_H)),
            _full(p['o_W1'].shape), _full((1, ENC_H)),
            _full(p['o_W2'].shape), _full((1, ENC_H)),
            _full(p['o_W3'].shape), _full((1, OUT_C)),
            _full((KCODE, OUT_C)),
        ],
        out_specs=[
            pl.BlockSpec((_BLK, OUT_C), lambda i: (i, 0)),
            _full((1, 128)),
        ],
        out_shape=[
            jax.ShapeDtypeStruct((N, OUT_C), jnp.float32),
            jax.ShapeDtypeStruct((1, 128), jnp.float32),
        ],
    )(x_save[0], x_save[1], x_AA,
      p['h_W1'], r2(p['h_b1']), p['h_W2'], r2(p['h_b2']),
      p['o_W1'], r2(p['o_b1']), p['o_W2'], r2(p['o_b2']),
      p['o_W3'], r2(p['o_b3']), p['codebook'])

    return z, loss_v[0, 0]


# speedup vs baseline: 12.0177x; 1.0806x over previous
import functools

import jax
import jax.numpy as jnp
from jax import lax
from jax.experimental import pallas as pl
from jax.experimental.pallas import tpu as pltpu
from jax.experimental.pallas import tpu_sc as plsc

_N_TILES = 16
_CH = 80
_NB = 4
_BLK = 2000
_EPS = 1e-5


def _gelu(x):
    return 0.5 * x * (1.0 + lax.erf(x * (2.0 ** -0.5)))


def _make_sc_agg(NPAD, F, E):
    EPT = E // _N_TILES
    NCH = EPT // _CH
    RPT = NPAD // _N_TILES
    mesh = plsc.VectorSubcoreMesh(core_axis_name="c", subcore_axis_name="s")

    NI = 2 * _NB
    out_type = jax.ShapeDtypeStruct((2, NPAD, F), jnp.float32)
    scratch = (
        [pltpu.VMEM((_CH,), jnp.int32) for _ in range(2 * NI)]
        + [pltpu.VMEM((_CH, F), jnp.float32) for _ in range(_NB)]
        + [pltpu.SemaphoreType.DMA for _ in range(NI)]
        + [pltpu.SemaphoreType.DMA for _ in range(2 * _NB)]
        + [pltpu.VMEM_SHARED((NPAD, F), jnp.float32)]
    )

    def body(h_hbm, eb_hbm, ec_hbm, z_hbm, out_sums, *rest):
        idx_s = rest[0:NI]
        idx_d = rest[NI:2 * NI]
        rows = rest[2 * NI:2 * NI + _NB]
        isem = rest[2 * NI + _NB:3 * NI + _NB]
        gsem = rest[3 * NI + _NB:3 * NI + 2 * _NB]
        ssem = rest[3 * NI + 2 * _NB:3 * NI + 3 * _NB]
        acc = rest[3 * NI + 3 * _NB]

        c = lax.axis_index("c")
        s = lax.axis_index("s")
        r0 = s * RPT
        pltpu.sync_copy(z_hbm.at[pl.ds(r0, RPT)], acc.at[pl.ds(r0, RPT)])
        plsc.subcore_barrier()

        j0 = s * NCH
        last = NCH - 1

        def i_fire(t, p):
            off = (jnp.minimum(t, last) + j0) * _CH

            @pl.when(c == 0)
            def _():
                pltpu.async_copy(eb_hbm.at[0, pl.ds(off, _CH)], idx_s[p],
                                 isem[p])
                pltpu.async_copy(eb_hbm.at[1, pl.ds(off, _CH)], idx_d[p],
                                 isem[p])

            @pl.when(c != 0)
            def _():
                pltpu.async_copy(ec_hbm.at[0, pl.ds(off, _CH)], idx_s[p],
                                 isem[p])
                pltpu.async_copy(ec_hbm.at[1, pl.ds(off, _CH)], idx_d[p],
                                 isem[p])

        def i_wait(t, p):
            off = (jnp.minimum(t, last) + j0) * _CH
            pltpu.make_async_copy(eb_hbm.at[0, pl.ds(off, _CH)], idx_s[p],
                                  isem[p]).wait()
            pltpu.make_async_copy(eb_hbm.at[1, pl.ds(off, _CH)], idx_d[p],
                                  isem[p]).wait()

        def g_fire(t, p, b):
            pltpu.async_copy(h_hbm.at[idx_s[p]], rows[b], gsem[b])

        def g_wait(t, p, b):
            pltpu.make_async_copy(h_hbm.at[idx_s[p]], rows[b],
                                  gsem[b]).wait()

        def s_fire(t, p, b):
            pltpu.async_copy(rows[b], acc.at[idx_d[p]], ssem[b],
                             add=True)

        def s_wait(t, p, b):
            pltpu.make_async_copy(rows[b], acc.at[idx_d[p]],
                                  ssem[b]).wait()

        L = _NB - 1

        def full_step(t, u):
            q, qL, p, p4 = (u % _NB, (u - L) % _NB, u % NI, (u + _NB) % NI)
            s_wait(t - _NB, (u - _NB) % NI, q)
            i_fire(t + _NB, p4)
            i_wait(t, p)
            g_fire(t, p, q)
            g_wait(t - L, (u - L) % NI, qL)
            s_fire(t - L, (u - L) % NI, qL)

        for u in range(_NB):
            i_fire(u, u % NI)
        for u in range(_NB):
            i_wait(u, u % NI)
            g_fire(u, u % NI, u % _NB)
            i_fire(u + _NB, (u + _NB) % NI)
            if u >= L:
                g_wait(u - L, (u - L) % NI, (u - L) % _NB)
                s_fire(u - L, (u - L) % NI, (u - L) % _NB)
        for u in range(_NB, NI):
            full_step(u, u)

        def main(T, carry):
            base = NI + T * NI
            for u in range(NI):
                full_step(base + u, u)
            return carry
        nmain = (NCH - NI) // NI
        lax.fori_loop(0, nmain, main, 0)

        for t in range(NCH - _NB, NI + nmain * NI):
            i_wait(last, (t % NI + _NB) % NI)
        for t in range(NI + nmain * NI, NCH):
            u = t % NI
            s_wait(t - _NB, (u - _NB) % NI, u % _NB)
            i_wait(t, u)
            g_fire(t, u, u % _NB)
            g_wait(t - L, (u - L) % NI, (u - L) % _NB)
            s_fire(t - L, (u - L) % NI, (u - L) % _NB)
        for t in range(NCH - L, NCH):
            u = t % NI
            g_wait(t, u, t % _NB)
            s_fire(t, u, t % _NB)
        for t in range(NCH - _NB, NCH):
            s_wait(t, t % NI, t % _NB)

        plsc.subcore_barrier()
        pltpu.sync_copy(acc.at[pl.ds(r0, RPT)],
                        out_sums.at[c, pl.ds(r0, RPT)])

    return pl.kernel(body, out_type=out_type, mesh=mesh,
                     scratch_types=tuple(scratch),
                     compiler_params=pltpu.CompilerParams(
                         use_tc_tiling_on_sc=False))


def _make_sc_counts(N, E):
    EPT = E // _N_TILES
    NCH = EPT // _CH
    RPTC = N // _N_TILES
    NI = 2 * _NB
    mesh = plsc.VectorSubcoreMesh(core_axis_name="c", subcore_axis_name="s")
    out_type = jax.ShapeDtypeStruct((2, N, 16), jnp.float32)
    scratch = (
        [pltpu.VMEM((_CH,), jnp.int32) for _ in range(NI)]
        + [pltpu.SemaphoreType.DMA for _ in range(NI)]
        + [pltpu.SemaphoreType.DMA for _ in range(NI)]
        + [pltpu.VMEM((_CH, 16), jnp.float32)]
        + [pltpu.VMEM_SHARED((N, 16), jnp.float32)]
    )

    def body(eb_hbm, ec_hbm, z_hbm, one_hbm, out_cnts, *rest):
        idx_d = rest[0:NI]
        isem = rest[NI:2 * NI]
        csem = rest[2 * NI:3 * NI]
        ones_v = rest[3 * NI]
        cnt = rest[3 * NI + 1]

        c = lax.axis_index("c")
        s = lax.axis_index("s")
        r0c = s * RPTC
        pltpu.sync_copy(z_hbm.at[pl.ds(r0c, RPTC), pl.ds(0, 16)],
                        cnt.at[pl.ds(r0c, RPTC)])
        pltpu.sync_copy(one_hbm, ones_v)
        plsc.subcore_barrier()

        j0 = s * NCH
        last = NCH - 1

        def i_fire(t, p):
            off = (jnp.minimum(t, last) + j0) * _CH

            @pl.when(c == 0)
            def _():
                pltpu.async_copy(eb_hbm.at[1, pl.ds(off, _CH)], idx_d[p],
                                 isem[p])

            @pl.when(c != 0)
            def _():
                pltpu.async_copy(ec_hbm.at[1, pl.ds(off, _CH)], idx_d[p],
                                 isem[p])

        def i_wait(t, p):
            off = (jnp.minimum(t, last) + j0) * _CH
            pltpu.make_async_copy(eb_hbm.at[1, pl.ds(off, _CH)], idx_d[p],
                                  isem[p]).wait()

        def c_fire(t, p):
            pltpu.async_copy(ones_v, cnt.at[idx_d[p]], csem[p],
                             add=True)

        def c_wait(t, p):
            pltpu.make_async_copy(ones_v, cnt.at[idx_d[p]],
                                  csem[p]).wait()

        def full_step(t, u):
            p, p4 = u % NI, (u + _NB) % NI
            c_wait(t - _NB, (u - _NB) % NI)
            i_fire(t + _NB, p4)
            i_wait(t, p)
            c_fire(t, p)

        for u in range(_NB):
            i_fire(u, u)
        for u in range(_NB):
            i_wait(u, u)
            c_fire(u, u)
            i_fire(u + _NB, u + _NB)
        for u in range(_NB, NI):
            full_step(u, u)

        def main(T, carry):
            base = NI + T * NI
            for u in range(NI):
                full_step(base + u, u)
            return carry
        nmain = (NCH - NI) // NI
        lax.fori_loop(0, nmain, main, 0)

        for t in range(NCH - _NB, NI + nmain * NI):
            i_wait(last, (t % NI + _NB) % NI)
        for t in range(NI + nmain * NI, NCH):
            u = t % NI
            c_wait(t - _NB, (u - _NB) % NI)
            i_wait(t, u)
            c_fire(t, u)
        for t in range(NCH - _NB, NCH):
            c_wait(t, t % NI)

        plsc.subcore_barrier()
        pltpu.sync_copy(cnt.at[pl.ds(r0c, RPTC)],
                        out_cnts.at[c, pl.ds(r0c, RPTC)])

    return pl.kernel(body, out_type=out_type, mesh=mesh,
                     scratch_types=tuple(scratch),
                     compiler_params=pltpu.CompilerParams(
                         use_tc_tiling_on_sc=False))


def _pre_body(x_ref, lnw, lnb, W1, b1, W2, b2, o_ref):
    x = x_ref[...]
    m = jnp.mean(x, -1, keepdims=True)
    v = jnp.mean((x - m) ** 2, -1, keepdims=True)
    h = (x - m) / jnp.sqrt(v + _EPS) * lnw[...] + lnb[...]
    h = _gelu(jnp.dot(h, W1[...], preferred_element_type=jnp.float32) + b1[...])
    h = _gelu(jnp.dot(h, W2[...], preferred_element_type=jnp.float32) + b2[...])
    o_ref[...] = h


def _hw_body(h_ref, Wr0, Wr1, o_ref):
    o_ref[...] = jnp.dot(h_ref[...], Wr0[...] + Wr1[...],
                         preferred_element_type=jnp.float32)


def _conv_body(hw_ref, sums_ref, cnts_ref, Wl0, bl0, Wl1, bl1,
               t_ref, st_ref):
    c0 = jnp.maximum(cnts_ref[0, :, 0:1], 1.0)
    c1 = jnp.maximum(cnts_ref[1, :, 0:1], 1.0)
    a0 = sums_ref[0] / c0
    a1 = sums_ref[1] / c1
    o = (jnp.dot(a0, Wl0[...], preferred_element_type=jnp.float32) + bl0[...]
         + jnp.dot(a1, Wl1[...], preferred_element_type=jnp.float32)
         + bl1[...] + hw_ref[...])
    t = _gelu(o * 0.5)
    t_ref[...] = t
    part = jnp.concatenate([jnp.sum(t, 0)[None], jnp.sum(t * t, 0)[None]], 0)

    @pl.when(pl.program_id(0) == 0)
    def _():
        st_ref[...] = jnp.zeros_like(st_ref)
    st_ref[...] += part


def _gn_body(n_rows, t_ref, st_ref, w, b, alpha, o_ref):
    t = t_ref[...]
    m = st_ref[0:1, :] / n_rows
    q = st_ref[1:2, :] / n_rows
    a = alpha[...]
    var = q - 2.0 * a * m * m + a * a * m * m
    o_ref[...] = w[...] * (t - m * a) / jnp.sqrt(var + _EPS) + b[...]


def _head_body(kcode, n_out, h1_ref, h2_ref, aa_ref, hW1, hb1, hW2, hb2,
               oW1, ob1, oW2, ob2, oW3, ob3, cb, z_ref, loss_ref):
    x = jnp.concatenate([h1_ref[...], h2_ref[...]], 1)
    x = _gelu(jnp.dot(x, hW1[...], preferred_element_type=jnp.float32) + hb1[...])
    x = _gelu(jnp.dot(x, hW2[...], preferred_element_type=jnp.float32) + hb2[...])
    x = jnp.concatenate([x, aa_ref[...]], 1)
    x = _gelu(jnp.dot(x, oW1[...], preferred_element_type=jnp.float32) + ob1[...])
    x = _gelu(jnp.dot(x, oW2[...], preferred_element_type=jnp.float32) + ob2[...])
    x = jnp.tanh(jnp.dot(x, oW3[...], preferred_element_type=jnp.float32) + ob3[...])
    cbv = cb[...]
    d = (jnp.sum(x * x, 1, keepdims=True)
         - 2.0 * lax.dot_general(x, cbv, (((1,), (1,)), ((), ())),
                                 preferred_element_type=jnp.float32)
         + jnp.sum(cbv * cbv, 1)[None, :])
    dmin = jnp.min(d, 1, keepdims=True)
    iota = lax.broadcasted_iota(jnp.int32, d.shape, 1)
    amin = jnp.min(jnp.where(d == dmin, iota, kcode), 1, keepdims=True)
    onehot = (iota == amin).astype(jnp.float32)
    zq = jnp.dot(onehot, cbv, preferred_element_type=jnp.float32)
    z_ref[...] = x + (zq - x)
    p = jnp.sum((zq - x) ** 2)

    first = pl.program_id(0) == 0
    last = pl.program_id(0) == pl.num_programs(0) - 1

    @pl.when(first)
    def _():
        loss_ref[...] = jnp.zeros_like(loss_ref)
    loss_ref[...] += p

    @pl.when(last)
    def _():
        loss_ref[...] *= 1.25 / n_out


def _full(shape):
    return pl.BlockSpec(shape, lambda i: tuple(0 for _ in shape))


def kernel(x_res, x_AA, edge_index_backbone, edge_index_contact, params):
    p = params
    N, IN_C = x_res.shape
    HID = p['in_W2'].shape[1]
    E = edge_index_backbone.shape[1]
    KCODE, OUT_C = p['codebook'].shape
    AA = x_AA.shape[1]
    ENC_H = p['h_W1'].shape[1]
    nblk = N // _BLK

    r2 = lambda a: a.reshape(1, -1)

    h = pl.pallas_call(
        _pre_body,
        grid=(nblk,),
        in_specs=[
            pl.BlockSpec((_BLK, IN_C), lambda i: (i, 0)),
            _full((1, IN_C)), _full((1, IN_C)),
            _full(p['in_W1'].shape), _full((1, p['in_W1'].shape[1])),
            _full(p['in_W2'].shape), _full((1, HID)),
        ],
        out_specs=pl.BlockSpec((_BLK, HID), lambda i: (i, 0)),
        out_shape=jax.ShapeDtypeStruct((N, HID), jnp.float32),
    )(x_res, r2(p['ln_w']), r2(p['ln_b']),
      p['in_W1'], r2(p['in_b1']), p['in_W2'], r2(p['in_b2']))

    eb, ec = edge_index_backbone, edge_index_contact
    NPAD = ((N + 8 * _N_TILES - 1) // (8 * _N_TILES)) * (8 * _N_TILES)
    zeros = jnp.zeros((NPAD, HID), jnp.float32)
    ones = jnp.ones((_CH, 16), jnp.float32)

    sc_agg = _make_sc_agg(NPAD, HID, E)
    sc_counts = _make_sc_counts(N, E)
    cnts = sc_counts(eb, ec, zeros, ones)

    hw_call = functools.partial(
        pl.pallas_call, _hw_body,
        grid=(nblk,),
        in_specs=[
            pl.BlockSpec((_BLK, HID), lambda i: (i, 0)),
            _full((HID, HID)), _full((HID, HID)),
        ],
        out_specs=pl.BlockSpec((_BLK, HID), lambda i: (i, 0)),
        out_shape=jax.ShapeDtypeStruct((N, HID), jnp.float32),
    )()

    conv_call = functools.partial(
        pl.pallas_call, _conv_body,
        grid=(nblk,),
        in_specs=[
            pl.BlockSpec((_BLK, HID), lambda i: (i, 0)),
            pl.BlockSpec((2, _BLK, HID), lambda i: (0, i, 0)),
            pl.BlockSpec((2, _BLK, 16), lambda i: (0, i, 0)),
            _full((HID, HID)), _full((1, HID)),
            _full((HID, HID)), _full((1, HID)),
        ],
        out_specs=[
            pl.BlockSpec((_BLK, HID), lambda i: (i, 0)),
            _full((2, HID)),
        ],
        out_shape=[
            jax.ShapeDtypeStruct((N, HID), jnp.float32),
            jax.ShapeDtypeStruct((2, HID), jnp.float32),
        ],
    )()

    gn_call = functools.partial(
        pl.pallas_call, functools.partial(_gn_body, float(N)),
        grid=(nblk,),
        in_specs=[
            pl.BlockSpec((_BLK, HID), lambda i: (i, 0)),
            _full((2, HID)),
            _full((1, HID)), _full((1, HID)), _full((1, HID)),
        ],
        out_specs=pl.BlockSpec((_BLK, HID), lambda i: (i, 0)),
        out_shape=jax.ShapeDtypeStruct((N, HID), jnp.float32),
    )()

    x_save = []
    for i in range(2):
        sums = sc_agg(h, eb, ec, zeros)
        cv = p['convs'][i]
        hw = hw_call(h, cv[0]['Wr'], cv[1]['Wr'])
        t, st = conv_call(
            hw, sums, cnts,
            cv[0]['Wl'], r2(cv[0]['bl']),
            cv[1]['Wl'], r2(cv[1]['bl']))
        g = p['gns'][i]
        h = gn_call(t, st, r2(g['w']), r2(g['b']), r2(g['alpha']))
        x_save.append(h)

    z, loss_v = pl.pallas_call(
        functools.partial(_head_body, KCODE, float(N * OUT_C)),
        grid=(nblk,),
        in_specs=[
            pl.BlockSpec((_BLK, HID), lambda i: (i, 0)),
            pl.BlockSpec((_BLK, HID), lambda i: (i, 0)),
            pl.BlockSpec((_BLK, AA), lambda i: (i, 0)),
            _full(p['h_W1'].shape), _full((1, ENC_H)),
            _full(p['h_W2'].shape), _full((1, ENC_H)),
            _full(p['o_W1'].shape), _full((1, ENC_H)),
            _full(p['o_W2'].shape), _full((1, ENC_H)),
            _full(p['o_W3'].shape), _full((1, OUT_C)),
            _full((KCODE, OUT_C)),
        ],
        out_specs=[
            pl.BlockSpec((_BLK, OUT_C), lambda i: (i, 0)),
            _full((1, 128)),
        ],
        out_shape=[
            jax.ShapeDtypeStruct((N, OUT_C), jnp.float32),
            jax.ShapeDtypeStruct((1, 128), jnp.float32),
        ],
    )(x_save[0], x_save[1], x_AA,
      p['h_W1'], r2(p['h_b1']), p['h_W2'], r2(p['h_b2']),
      p['o_W1'], r2(p['o_b1']), p['o_W2'], r2(p['o_b2']),
      p['o_W3'], r2(p['o_b3']), p['codebook'])

    return z, loss_v[0, 0]
